# Initial kernel scaffold; baseline (speedup 1.0000x reference)
import jax
import jax.numpy as jnp
from jax.experimental import pallas as pl


def kernel(x, som, class_count):
    raise NotImplementedError("write your pallas kernel here")


# speedup vs baseline: 2.4060x; 2.4060x over previous
import functools

import jax
import jax.numpy as jnp
from jax import lax
from jax.experimental import pallas as pl
from jax.experimental.pallas import tpu as pltpu


def _tc_body(x_ref, w_ref, cc_ref, sims_ref, bmu_ref, bpmi_ref,
             wn_ref, pmi_ref, *, uy):
    i = pl.program_id(0)

    @pl.when(i == 0)
    def _prep():
        wv = w_ref[...]
        wn = jnp.sqrt(jnp.sum(wv * wv, axis=1, keepdims=True))
        wn_ref[...] = wv / (wn + 1e-6)

        cc = cc_ref[...]
        denom = jnp.sum(cc, axis=1, keepdims=True)
        cond = cc / (denom + 1e-6)
        prior = jnp.sum(cc, axis=0, keepdims=True)
        prior = prior / (jnp.sum(cc) + 1e-6)
        pmi_ref[...] = jnp.log(cond / (prior + 1e-6) + 1e-6)

    xv = x_ref[...]
    xn = jnp.sqrt(jnp.sum(xv * xv, axis=1, keepdims=True))
    xv = xv / (xn + 1e-6)

    sims = lax.dot_general(
        xv, wn_ref[...],
        dimension_numbers=(((1,), (1,)), ((), ())),
        preferred_element_type=jnp.float32,
    )
    sims_ref[...] = sims

    tq, k = sims.shape
    flat = jnp.argmax(sims, axis=1).astype(jnp.int32)
    flat2 = flat.reshape(tq, 1)
    bmu_ref[...] = jnp.concatenate([flat2 // uy, flat2 % uy], axis=1)

    col = lax.broadcasted_iota(jnp.int32, (tq, k), 1)
    onehot = (col == flat2).astype(jnp.float32)
    bpmi_ref[...] = lax.dot_general(
        onehot, pmi_ref[...],
        dimension_numbers=(((1,), (0,)), ((), ())),
        preferred_element_type=jnp.float32,
    )


def kernel(x, som, class_count):
    q, d = x.shape
    ux, uy, _ = som.shape
    k = ux * uy
    c = class_count.shape[-1]
    w = som.reshape(k, d)
    cc = class_count.reshape(k, c)

    tq = 256
    grid = (q // tq,)

    sims, bmu, bmu_pmi = pl.pallas_call(
        functools.partial(_tc_body, uy=uy),
        grid=grid,
        in_specs=[
            pl.BlockSpec((tq, d), lambda i: (i, 0)),
            pl.BlockSpec((k, d), lambda i: (0, 0)),
            pl.BlockSpec((k, c), lambda i: (0, 0)),
        ],
        out_specs=[
            pl.BlockSpec((tq, k), lambda i: (i, 0)),
            pl.BlockSpec((tq, 2), lambda i: (i, 0)),
            pl.BlockSpec((tq, c), lambda i: (i, 0)),
        ],
        out_shape=[
            jax.ShapeDtypeStruct((q, k), jnp.float32),
            jax.ShapeDtypeStruct((q, 2), jnp.int32),
            jax.ShapeDtypeStruct((q, c), jnp.float32),
        ],
        scratch_shapes=[
            pltpu.VMEM((k, d), jnp.float32),
            pltpu.VMEM((k, c), jnp.float32),
        ],
    )(x, w, cc)

    return sims, bmu, bmu_pmi
